# Initial kernel scaffold; baseline (speedup 1.0000x reference)
import jax
import jax.numpy as jnp
from jax.experimental import pallas as pl


def kernel(action, table):
    raise NotImplementedError("write your pallas kernel here")


# speedup vs baseline: 2.4217x; 2.4217x over previous
import functools

import jax
import jax.numpy as jnp
from jax import lax
from jax.experimental import pallas as pl
from jax.experimental.pallas import tpu as pltpu
from jax.experimental.pallas import tpu_sc as plsc

DIM = 16
NW = 32
CH = 2048
GB = 128


def _emb_call(n_idx):
    per_w = n_idx // NW
    n_chunks = per_w // CH
    mesh = plsc.VectorSubcoreMesh(core_axis_name="c", subcore_axis_name="s")

    @functools.partial(
        pl.kernel,
        mesh=mesh,
        out_type=jax.ShapeDtypeStruct((n_idx, DIM), jnp.float32),
        scratch_types=[
            pltpu.VMEM((CH,), jnp.int32),
            pltpu.VMEM((CH, DIM), jnp.float32),
            pltpu.SemaphoreType.DMA,
        ],
        compiler_params=pltpu.CompilerParams(use_tc_tiling_on_sc=False),
    )
    def emb(idx_hbm, table_hbm, out_hbm, idxbuf, rowbuf, gsem):
        wid = lax.axis_index("s") * 2 + lax.axis_index("c")
        base = wid * per_w

        def chunk_body(ci, carry):
            off = base + ci * CH
            pltpu.sync_copy(idx_hbm.at[pl.ds(off, CH)], idxbuf)

            def add_body(k, c):
                sl = pl.ds(k * 16, 16)
                idxbuf[sl] = idxbuf[sl] + 1
                return c

            lax.fori_loop(0, CH // 16, add_body, 0)

            copies = []
            for j in range(CH // GB):
                copies.append(pltpu.async_copy(
                    table_hbm.at[idxbuf.at[pl.ds(j * GB, GB)]],
                    rowbuf.at[pl.ds(j * GB, GB)],
                    gsem,
                ))
            for c in copies:
                c.wait()

            pltpu.sync_copy(rowbuf, out_hbm.at[pl.ds(off, CH)])
            return carry

        lax.fori_loop(0, n_chunks, chunk_body, 0)

    return emb


def kernel(action, table):
    B, L, _ = action.shape
    n_idx = B * L
    idx = action.reshape(n_idx)
    out = _emb_call(n_idx)(idx, table)
    return out.reshape(B, L, DIM)


# speedup vs baseline: 2.4910x; 1.0286x over previous
import functools

import jax
import jax.numpy as jnp
from jax import lax
from jax.experimental import pallas as pl
from jax.experimental.pallas import tpu as pltpu
from jax.experimental.pallas import tpu_sc as plsc

DIM = 16
NW = 32
GB = 128
CR = 16


def _emb_call(n_rows):
    rows_per_w = n_rows // NW
    n_chunks = rows_per_w // CR
    mesh = plsc.VectorSubcoreMesh(core_axis_name="c", subcore_axis_name="s")

    @functools.partial(
        pl.kernel,
        mesh=mesh,
        out_type=jax.ShapeDtypeStruct((n_rows, GB, DIM), jnp.float32),
        scratch_types=[
            pltpu.VMEM((CR, GB), jnp.int32),
            pltpu.VMEM((2, CR, GB, DIM), jnp.float32),
            pltpu.SemaphoreType.DMA,
            pltpu.SemaphoreType.DMA,
            pltpu.SemaphoreType.DMA,
        ],
        compiler_params=pltpu.CompilerParams(use_tc_tiling_on_sc=False),
    )
    def emb(idx_hbm, table_hbm, out_hbm, idxbuf, rowbuf, gsem, osem0, osem1):
        wid = lax.axis_index("s") * 2 + lax.axis_index("c")
        row0 = wid * rows_per_w
        osems = (osem0, osem1)

        def half_body(ci, b):
            r0 = row0 + ci * CR
            pltpu.sync_copy(idx_hbm.at[pl.ds(r0, CR)], idxbuf)

            def add_body(i, c):
                for s in range(GB // 16):
                    sl = pl.ds(s * 16, 16)
                    idxbuf[i, sl] = idxbuf[i, sl] + 1
                return c

            lax.fori_loop(0, CR, add_body, 0)

            @pl.when(ci >= 2)
            def _():
                pltpu.make_async_copy(
                    rowbuf.at[b], out_hbm.at[pl.ds(r0 - 2 * CR, CR)], osems[b]
                ).wait()

            copies = []
            for j in range(CR):
                copies.append(pltpu.async_copy(
                    table_hbm.at[idxbuf.at[j]], rowbuf.at[b].at[j], gsem
                ))
            for c in copies:
                c.wait()
            pltpu.async_copy(
                rowbuf.at[b], out_hbm.at[pl.ds(r0, CR)], osems[b]
            )
            return b

        def chunk_pair(g, carry):
            half_body(2 * g, 0)
            half_body(2 * g + 1, 1)
            return carry

        lax.fori_loop(0, n_chunks // 2, chunk_pair, 0)

        for b in range(2):
            r0 = row0 + (n_chunks - 2 + b) * CR
            pltpu.make_async_copy(
                rowbuf.at[b], out_hbm.at[pl.ds(r0, CR)], osems[b]
            ).wait()

    return emb


def kernel(action, table):
    B, L, _ = action.shape
    n_idx = B * L
    idx = action.reshape(n_idx // GB, GB)
    out = _emb_call(n_idx // GB)(idx, table)
    return out.reshape(B, L, DIM)
